# baseline (speedup 1.0000x reference)
import functools

import jax
import jax.numpy as jnp
from jax import lax
from jax.experimental import pallas as pl
from jax.experimental.pallas import tpu as pltpu
from jax.experimental.pallas import tpu_sc as plsc

N = 10000
D = 128
H = 128
E = 320000

NC = 2
NS = 16
NW = NC * NS

CH = 128
CPT = 80
NBUF = 2
GBLK = 16
EPT = CPT * CH
E_PAD = NW * EPT
ACC_ROWS = 10240
ROWS_PER_TILE = ACC_ROWS // NS
N_PAD = ACC_ROWS


def _zero_f32_vmem2(ref, n_rows, n_cols):
    z = jnp.zeros((16,), jnp.float32)

    def body(r, _):
        for c in range(n_cols // 16):
            ref[r, pl.ds(c * 16, 16)] = z
        return 0

    lax.fori_loop(0, n_rows, body, 0)


def _zero_f32_vmem1(ref, n):
    z = jnp.zeros((16,), jnp.float32)

    def body(i, _):
        ref[pl.ds(i * 16, 16)] = z
        return 0

    lax.fori_loop(0, n // 16, body, 0)


def _deg_body(dst_hbm, out_hbm, acc, ones_v, didx2, zbuf_v):
    c = lax.axis_index("c")
    s = lax.axis_index("s")
    wid = c * NS + s

    _zero_f32_vmem1(zbuf_v, ROWS_PER_TILE)
    pltpu.sync_copy(zbuf_v, acc.at[pl.ds(s * ROWS_PER_TILE, ROWS_PER_TILE)])
    o = jnp.full((16,), 1.0, jnp.float32)
    for c16 in range(CH // 16):
        ones_v[pl.ds(c16 * 16, 16)] = o
    pltpu.sync_copy(dst_hbm.at[wid], didx2)
    plsc.subcore_barrier()

    def chunk(k, _):
        pltpu.sync_copy(ones_v, acc.at[didx2.at[k]], add=True)
        return 0

    lax.fori_loop(0, CPT, chunk, 0)
    plsc.subcore_barrier()

    pltpu.sync_copy(acc.at[pl.ds(s * ROWS_PER_TILE, ROWS_PER_TILE)],
                    out_hbm.at[c, pl.ds(s * ROWS_PER_TILE, ROWS_PER_TILE)])


def _agg_body(g_hbm, srch_hbm, dst_hbm, out_hbm, acc, rows2, sidxb, didxb,
              sem00, sem01, sem10, sem11):
    c = lax.axis_index("c")
    s = lax.axis_index("s")
    wid = c * NS + s
    sems = [[sem00, sem01], [sem10, sem11]]
    HCH = CH // 2

    z16 = jnp.zeros((16,), jnp.float32)

    def zbody(r, _):
        for cc in range(H // 16):
            rows2[0, r, pl.ds(cc * 16, 16)] = z16
        return 0

    lax.fori_loop(0, CH, zbody, 0)
    for z in range(ROWS_PER_TILE // CH):
        pltpu.sync_copy(rows2.at[0], acc.at[pl.ds(s * ROWS_PER_TILE + z * CH, CH)])
    plsc.subcore_barrier()

    def issue(kk, b):
        for h in range(2):
            pltpu.async_copy(g_hbm.at[sidxb.at[2 * kk + h]],
                             rows2.at[b, pl.ds(h * HCH, HCH)], sems[b][h])

    def drain(kk, b):
        for h in range(2):
            pltpu.make_async_copy(g_hbm.at[sidxb.at[2 * kk + h]],
                                  rows2.at[b, pl.ds(h * HCH, HCH)],
                                  sems[b][h]).wait()

    def block(j, _):
        pltpu.sync_copy(srch_hbm.at[wid, pl.ds(j * 2 * GBLK, 2 * GBLK)], sidxb)
        pltpu.sync_copy(dst_hbm.at[wid, pl.ds(j * GBLK, GBLK)], didxb)
        issue(0, 0)

        def pair(i, _):
            for b in range(NBUF):
                k = NBUF * i + b
                kn = k + 1
                bn = (b + 1) % NBUF

                @pl.when(kn < GBLK)
                def _():
                    issue(kn, bn)

                drain(k, b)
                pltpu.sync_copy(rows2.at[b], acc.at[didxb.at[k]], add=True)
            return 0

        lax.fori_loop(0, GBLK // NBUF, pair, 0)
        return 0

    lax.fori_loop(0, CPT // GBLK, block, 0)
    plsc.subcore_barrier()

    pltpu.sync_copy(acc.at[pl.ds(s * ROWS_PER_TILE, ROWS_PER_TILE)],
                    out_hbm.at[c, pl.ds(s * ROWS_PER_TILE, ROWS_PER_TILE)])


def _scale_body(x_ref, w1_ref, deg_ref, g_ref):
    i = pl.program_id(0)
    bn = g_ref.shape[0]
    deg = (deg_ref[0, pl.ds(i * bn, bn)] + deg_ref[1, pl.ds(i * bn, bn)] + 1.0)
    dinv = 1.0 / jnp.sqrt(deg)
    h = lax.dot_general(x_ref[...], w1_ref[...], (((1,), (1,)), ((), ())),
                        preferred_element_type=jnp.float32)
    g_ref[...] = h * dinv[:, None]


def _head_body(agg_ref, g_ref, deg_ref, b1_ref, wl_ref, bl_ref, y_ref):
    i = pl.program_id(0)
    bn = g_ref.shape[0]
    deg = (deg_ref[0, pl.ds(i * bn, bn)] + deg_ref[1, pl.ds(i * bn, bn)] + 1.0)
    dinv = 1.0 / jnp.sqrt(deg)
    tot = agg_ref[0] + agg_ref[1] + g_ref[...]
    conv = tot * dinv[:, None] + b1_ref[0, :][None, :]
    conv = jnp.maximum(conv, 0.0)
    y = lax.dot_general(conv, wl_ref[...], (((1,), (0,)), ((), ())),
                        preferred_element_type=jnp.float32)
    y_ref[...] = y + bl_ref[0, 0]


def kernel(x, edge_index, W1, b1, Wl, bl):
    src = edge_index[0]
    dst = edge_index[1]
    pad = E_PAD - E
    ar = jnp.arange(pad, dtype=jnp.int32)
    src_p = jnp.concatenate([src, ar % N]).reshape(NW, CPT, CH)
    dst_p = jnp.concatenate([dst, N + ar % (ACC_ROWS - N)]).reshape(NW, CPT, CH)

    mesh = plsc.VectorSubcoreMesh(core_axis_name="c", subcore_axis_name="s")

    deg_parts = pl.kernel(
        _deg_body,
        out_type=jax.ShapeDtypeStruct((NC, ACC_ROWS), jnp.float32),
        mesh=mesh,
        scratch_types=[
            pltpu.VMEM_SHARED((ACC_ROWS,), jnp.float32),
            pltpu.VMEM((CH,), jnp.float32),
            pltpu.VMEM((CPT, CH), jnp.int32),
            pltpu.VMEM((ROWS_PER_TILE,), jnp.float32),
        ],
        name="gcn_deg_sc",
    )(dst_p)

    deg2 = deg_parts

    BN = 2048
    NB = (N + BN - 1) // BN
    g = pl.pallas_call(
        _scale_body,
        grid=(NB,),
        in_specs=[
            pl.BlockSpec((BN, D), lambda i: (i, 0)),
            pl.BlockSpec((H, D), lambda i: (0, 0)),
            pl.BlockSpec((NC, N_PAD), lambda i: (0, 0)),
        ],
        out_specs=pl.BlockSpec((BN, H), lambda i: (i, 0)),
        out_shape=jax.ShapeDtypeStruct((N, H), jnp.float32),
        name="gcn_scale_tc",
    )(x, W1, deg2)

    agg_parts = pl.kernel(
        _agg_body,
        out_type=jax.ShapeDtypeStruct((NC, N_PAD, H), jnp.float32),
        mesh=mesh,
        scratch_types=[
            pltpu.VMEM_SHARED((ACC_ROWS, H), jnp.float32),
            pltpu.VMEM((NBUF, CH, H), jnp.float32),
            pltpu.VMEM((2 * GBLK, CH // 2), jnp.int32),
            pltpu.VMEM((GBLK, CH), jnp.int32),
            pltpu.SemaphoreType.DMA,
            pltpu.SemaphoreType.DMA,
            pltpu.SemaphoreType.DMA,
            pltpu.SemaphoreType.DMA,
        ],
        name="gcn_agg_sc",
    )(g, src_p.reshape(NW, 2 * CPT, CH // 2), dst_p)

    y = pl.pallas_call(
        _head_body,
        grid=(NB,),
        in_specs=[
            pl.BlockSpec((NC, BN, H), lambda i: (0, i, 0)),
            pl.BlockSpec((BN, H), lambda i: (i, 0)),
            pl.BlockSpec((NC, N_PAD), lambda i: (0, 0)),
            pl.BlockSpec((1, H), lambda i: (0, 0)),
            pl.BlockSpec((H, 1), lambda i: (0, 0)),
            pl.BlockSpec((1, 1), lambda i: (0, 0)),
        ],
        out_specs=pl.BlockSpec((BN, 1), lambda i: (i, 0)),
        out_shape=jax.ShapeDtypeStruct((N, 1), jnp.float32),
        name="gcn_head_tc",
    )(agg_parts, g, deg2, b1.reshape(1, H), Wl.reshape(H, 1), bl.reshape(1, 1))

    return y[:, 0]


# speedup vs baseline: 1.0135x; 1.0135x over previous
import functools

import jax
import jax.numpy as jnp
from jax import lax
from jax.experimental import pallas as pl
from jax.experimental.pallas import tpu as pltpu
from jax.experimental.pallas import tpu_sc as plsc

N = 10000
D = 128
H = 128
E = 320000

NC = 2
NS = 16
NW = NC * NS

CH = 128
CPT = 80
NBUF = 2
GBLK = 16
EPT = CPT * CH
E_PAD = NW * EPT
ACC_ROWS = 10240
ROWS_PER_TILE = ACC_ROWS // NS
N_PAD = ACC_ROWS


def _zero_f32_vmem2(ref, n_rows, n_cols):
    z = jnp.zeros((16,), jnp.float32)

    def body(r, _):
        for c in range(n_cols // 16):
            ref[r, pl.ds(c * 16, 16)] = z
        return 0

    lax.fori_loop(0, n_rows, body, 0)


def _zero_f32_vmem1(ref, n):
    z = jnp.zeros((16,), jnp.float32)

    def body(i, _):
        ref[pl.ds(i * 16, 16)] = z
        return 0

    lax.fori_loop(0, n // 16, body, 0)


def _deg_body(dst_hbm, out_hbm, acc, ones_v, didx2, zbuf_v):
    c = lax.axis_index("c")
    s = lax.axis_index("s")
    wid = c * NS + s

    _zero_f32_vmem1(zbuf_v, ROWS_PER_TILE)
    pltpu.sync_copy(zbuf_v, acc.at[pl.ds(s * ROWS_PER_TILE, ROWS_PER_TILE)])
    o = jnp.full((16,), 1.0, jnp.float32)
    for c16 in range(CH // 16):
        ones_v[pl.ds(c16 * 16, 16)] = o
    pltpu.sync_copy(dst_hbm.at[wid], didx2)
    plsc.subcore_barrier()

    def chunk(k, _):
        pltpu.sync_copy(ones_v, acc.at[didx2.at[k]], add=True)
        return 0

    lax.fori_loop(0, CPT, chunk, 0)
    plsc.subcore_barrier()

    pltpu.sync_copy(acc.at[pl.ds(s * ROWS_PER_TILE, ROWS_PER_TILE)],
                    out_hbm.at[c, pl.ds(s * ROWS_PER_TILE, ROWS_PER_TILE)])


def _agg_body(g_hbm, srch_hbm, dst_hbm, out_hbm, acc, rows2, sidxb, didxb,
              sem00, sem01, sem10, sem11):
    c = lax.axis_index("c")
    s = lax.axis_index("s")
    wid = c * NS + s
    sems = [[sem00, sem01], [sem10, sem11]]
    HCH = CH // 2

    z16 = jnp.zeros((16,), jnp.float32)

    def zbody(r, _):
        for cc in range(H // 16):
            rows2[0, r, pl.ds(cc * 16, 16)] = z16
        return 0

    lax.fori_loop(0, CH, zbody, 0)
    for z in range(ROWS_PER_TILE // CH):
        pltpu.sync_copy(rows2.at[0], acc.at[pl.ds(s * ROWS_PER_TILE + z * CH, CH)])
    plsc.subcore_barrier()

    def issue(kk, b):
        pltpu.async_copy(g_hbm.at[sidxb.at[kk]], rows2.at[b], sems[b][0])

    def drain(kk, b):
        pltpu.make_async_copy(g_hbm.at[sidxb.at[kk]], rows2.at[b],
                              sems[b][0]).wait()

    def block(j, _):
        pltpu.sync_copy(srch_hbm.at[wid, pl.ds(j * GBLK, GBLK)], sidxb)
        pltpu.sync_copy(dst_hbm.at[wid, pl.ds(j * GBLK, GBLK)], didxb)
        issue(0, 0)

        def pair(i, _):
            for b in range(NBUF):
                k = NBUF * i + b
                kn = k + 1
                bn = (b + 1) % NBUF

                @pl.when(kn < GBLK)
                def _():
                    issue(kn, bn)

                drain(k, b)
                pltpu.sync_copy(rows2.at[b], acc.at[didxb.at[k]], add=True)
            return 0

        lax.fori_loop(0, GBLK // NBUF, pair, 0)
        return 0

    lax.fori_loop(0, CPT // GBLK, block, 0)
    plsc.subcore_barrier()

    pltpu.sync_copy(acc.at[pl.ds(s * ROWS_PER_TILE, ROWS_PER_TILE)],
                    out_hbm.at[c, pl.ds(s * ROWS_PER_TILE, ROWS_PER_TILE)])


def _scale_body(x_ref, w1_ref, deg_ref, g_ref):
    i = pl.program_id(0)
    bn = g_ref.shape[0]
    deg = (deg_ref[0, pl.ds(i * bn, bn)] + deg_ref[1, pl.ds(i * bn, bn)] + 1.0)
    dinv = 1.0 / jnp.sqrt(deg)
    h = lax.dot_general(x_ref[...], w1_ref[...], (((1,), (1,)), ((), ())),
                        preferred_element_type=jnp.float32)
    g_ref[...] = h * dinv[:, None]


def _head_body(agg_ref, g_ref, deg_ref, b1_ref, wl_ref, bl_ref, y_ref):
    i = pl.program_id(0)
    bn = g_ref.shape[0]
    deg = (deg_ref[0, pl.ds(i * bn, bn)] + deg_ref[1, pl.ds(i * bn, bn)] + 1.0)
    dinv = 1.0 / jnp.sqrt(deg)
    tot = agg_ref[0] + agg_ref[1] + g_ref[...]
    conv = tot * dinv[:, None] + b1_ref[0, :][None, :]
    conv = jnp.maximum(conv, 0.0)
    y = lax.dot_general(conv, wl_ref[...], (((1,), (0,)), ((), ())),
                        preferred_element_type=jnp.float32)
    y_ref[...] = y + bl_ref[0, 0]


def kernel(x, edge_index, W1, b1, Wl, bl):
    src = edge_index[0]
    dst = edge_index[1]
    pad = E_PAD - E
    ar = jnp.arange(pad, dtype=jnp.int32)
    src_p = jnp.concatenate([src, ar % N]).reshape(NW, CPT, CH)
    dst_p = jnp.concatenate([dst, N + ar % (ACC_ROWS - N)]).reshape(NW, CPT, CH)

    mesh = plsc.VectorSubcoreMesh(core_axis_name="c", subcore_axis_name="s")

    deg_parts = pl.kernel(
        _deg_body,
        out_type=jax.ShapeDtypeStruct((NC, ACC_ROWS), jnp.float32),
        mesh=mesh,
        scratch_types=[
            pltpu.VMEM_SHARED((ACC_ROWS,), jnp.float32),
            pltpu.VMEM((CH,), jnp.float32),
            pltpu.VMEM((CPT, CH), jnp.int32),
            pltpu.VMEM((ROWS_PER_TILE,), jnp.float32),
        ],
        name="gcn_deg_sc",
    )(dst_p)

    deg2 = deg_parts

    BN = 2048
    NB = (N + BN - 1) // BN
    g = pl.pallas_call(
        _scale_body,
        grid=(NB,),
        in_specs=[
            pl.BlockSpec((BN, D), lambda i: (i, 0)),
            pl.BlockSpec((H, D), lambda i: (0, 0)),
            pl.BlockSpec((NC, N_PAD), lambda i: (0, 0)),
        ],
        out_specs=pl.BlockSpec((BN, H), lambda i: (i, 0)),
        out_shape=jax.ShapeDtypeStruct((N, H), jnp.float32),
        name="gcn_scale_tc",
    )(x, W1, deg2)

    agg_parts = pl.kernel(
        _agg_body,
        out_type=jax.ShapeDtypeStruct((NC, N_PAD, H), jnp.float32),
        mesh=mesh,
        scratch_types=[
            pltpu.VMEM_SHARED((ACC_ROWS, H), jnp.float32),
            pltpu.VMEM((NBUF, CH, H), jnp.float32),
            pltpu.VMEM((GBLK, CH), jnp.int32),
            pltpu.VMEM((GBLK, CH), jnp.int32),
            pltpu.SemaphoreType.DMA,
            pltpu.SemaphoreType.DMA,
            pltpu.SemaphoreType.DMA,
            pltpu.SemaphoreType.DMA,
        ],
        name="gcn_agg_sc",
    )(g, src_p, dst_p)

    y = pl.pallas_call(
        _head_body,
        grid=(NB,),
        in_specs=[
            pl.BlockSpec((NC, BN, H), lambda i: (0, i, 0)),
            pl.BlockSpec((BN, H), lambda i: (i, 0)),
            pl.BlockSpec((NC, N_PAD), lambda i: (0, 0)),
            pl.BlockSpec((1, H), lambda i: (0, 0)),
            pl.BlockSpec((H, 1), lambda i: (0, 0)),
            pl.BlockSpec((1, 1), lambda i: (0, 0)),
        ],
        out_specs=pl.BlockSpec((BN, 1), lambda i: (i, 0)),
        out_shape=jax.ShapeDtypeStruct((N, 1), jnp.float32),
        name="gcn_head_tc",
    )(agg_parts, g, deg2, b1.reshape(1, H), Wl.reshape(H, 1), bl.reshape(1, 1))

    return y[:, 0]


# speedup vs baseline: 1.0895x; 1.0750x over previous
import functools

import jax
import jax.numpy as jnp
from jax import lax
from jax.experimental import pallas as pl
from jax.experimental.pallas import tpu as pltpu
from jax.experimental.pallas import tpu_sc as plsc

N = 10000
D = 128
H = 128
E = 320000

NC = 2
NS = 16
NW = NC * NS

CH = 128
CPT = 80
NBUF = 2
GBLK = 16
EPT = CPT * CH
E_PAD = NW * EPT
ACC_ROWS = 10240
ROWS_PER_TILE = ACC_ROWS // NS
N_PAD = ACC_ROWS


def _zero_f32_vmem2(ref, n_rows, n_cols):
    z = jnp.zeros((16,), jnp.float32)

    def body(r, _):
        for c in range(n_cols // 16):
            ref[r, pl.ds(c * 16, 16)] = z
        return 0

    lax.fori_loop(0, n_rows, body, 0)


def _zero_f32_vmem1(ref, n):
    z = jnp.zeros((16,), jnp.float32)

    def body(i, _):
        ref[pl.ds(i * 16, 16)] = z
        return 0

    lax.fori_loop(0, n // 16, body, 0)


def _deg_body(ei_hbm, out_hbm, acc, ones_v, didx2, zbuf_v):
    c = lax.axis_index("c")
    s = lax.axis_index("s")
    wid = c * NS + s

    _zero_f32_vmem1(zbuf_v, ROWS_PER_TILE)
    pltpu.sync_copy(zbuf_v, acc.at[pl.ds(s * ROWS_PER_TILE, ROWS_PER_TILE)])
    o = jnp.full((16,), 1.0, jnp.float32)
    for c16 in range(CH // 16):
        ones_v[pl.ds(c16 * 16, 16)] = o
    pltpu.sync_copy(ei_hbm.at[1, wid], didx2)
    plsc.subcore_barrier()

    def chunk(k, _):
        pltpu.sync_copy(ones_v, acc.at[didx2.at[k]], add=True)
        return 0

    lax.fori_loop(0, CPT, chunk, 0)
    plsc.subcore_barrier()

    pltpu.sync_copy(acc.at[pl.ds(s * ROWS_PER_TILE, ROWS_PER_TILE)],
                    out_hbm.at[c, pl.ds(s * ROWS_PER_TILE, ROWS_PER_TILE)])


def _agg_body(g_hbm, ei_hbm, out_hbm, acc, rows2, sidxb, didxb,
              sem00, sem01, sem10, sem11):
    c = lax.axis_index("c")
    s = lax.axis_index("s")
    wid = c * NS + s
    sems = [[sem00, sem01], [sem10, sem11]]
    HCH = CH // 2

    z16 = jnp.zeros((16,), jnp.float32)

    def zbody(r, _):
        for cc in range(H // 16):
            rows2[0, r, pl.ds(cc * 16, 16)] = z16
        return 0

    lax.fori_loop(0, CH, zbody, 0)
    for z in range(ROWS_PER_TILE // CH):
        pltpu.sync_copy(rows2.at[0], acc.at[pl.ds(s * ROWS_PER_TILE + z * CH, CH)])
    plsc.subcore_barrier()

    def issue(kk, b):
        pltpu.async_copy(g_hbm.at[sidxb.at[kk]], rows2.at[b], sems[b][0])

    def drain(kk, b):
        pltpu.make_async_copy(g_hbm.at[sidxb.at[kk]], rows2.at[b],
                              sems[b][0]).wait()

    def block(j, _):
        pltpu.sync_copy(ei_hbm.at[0, wid, pl.ds(j * GBLK, GBLK)], sidxb)
        pltpu.sync_copy(ei_hbm.at[1, wid, pl.ds(j * GBLK, GBLK)], didxb)
        issue(0, 0)

        def pair(i, _):
            for b in range(NBUF):
                k = NBUF * i + b
                kn = k + 1
                bn = (b + 1) % NBUF

                @pl.when(kn < GBLK)
                def _():
                    issue(kn, bn)

                drain(k, b)
                pltpu.sync_copy(rows2.at[b], acc.at[didxb.at[k]], add=True)
            return 0

        lax.fori_loop(0, GBLK // NBUF, pair, 0)
        return 0

    lax.fori_loop(0, CPT // GBLK, block, 0)
    plsc.subcore_barrier()

    pltpu.sync_copy(acc.at[pl.ds(s * ROWS_PER_TILE, ROWS_PER_TILE)],
                    out_hbm.at[c, pl.ds(s * ROWS_PER_TILE, ROWS_PER_TILE)])


def _scale_body(x_ref, w1_ref, deg_ref, g_ref):
    i = pl.program_id(0)
    bn = g_ref.shape[0]
    deg = (deg_ref[0, pl.ds(i * bn, bn)] + deg_ref[1, pl.ds(i * bn, bn)] + 1.0)
    dinv = 1.0 / jnp.sqrt(deg)
    h = lax.dot_general(x_ref[...], w1_ref[...], (((1,), (1,)), ((), ())),
                        preferred_element_type=jnp.float32)
    g_ref[...] = h * dinv[:, None]


def _head_body(agg_ref, g_ref, deg_ref, b1_ref, wl_ref, bl_ref, y_ref):
    i = pl.program_id(0)
    bn = g_ref.shape[0]
    deg = (deg_ref[0, pl.ds(i * bn, bn)] + deg_ref[1, pl.ds(i * bn, bn)] + 1.0)
    dinv = 1.0 / jnp.sqrt(deg)
    tot = agg_ref[0] + agg_ref[1] + g_ref[...]
    conv = tot * dinv[:, None] + b1_ref[0, :][None, :]
    conv = jnp.maximum(conv, 0.0)
    y = lax.dot_general(wl_ref[...], conv, (((1,), (1,)), ((), ())),
                        preferred_element_type=jnp.float32)
    y_ref[...] = y + bl_ref[0, 0]


def kernel(x, edge_index, W1, b1, Wl, bl):
    pad = E_PAD - E
    ar = jnp.arange(pad, dtype=jnp.int32)
    pad_pair = jnp.stack([ar % N, N + ar % (ACC_ROWS - N)])
    ei_p = jnp.concatenate([edge_index, pad_pair], axis=1)
    ei_p = ei_p.reshape(2, NW, CPT, CH)

    mesh = plsc.VectorSubcoreMesh(core_axis_name="c", subcore_axis_name="s")

    deg_parts = pl.kernel(
        _deg_body,
        out_type=jax.ShapeDtypeStruct((NC, ACC_ROWS), jnp.float32),
        mesh=mesh,
        scratch_types=[
            pltpu.VMEM_SHARED((ACC_ROWS,), jnp.float32),
            pltpu.VMEM((CH,), jnp.float32),
            pltpu.VMEM((CPT, CH), jnp.int32),
            pltpu.VMEM((ROWS_PER_TILE,), jnp.float32),
        ],
        name="gcn_deg_sc",
    )(ei_p)

    deg2 = deg_parts

    BN = 2048
    NB = (N + BN - 1) // BN
    g = pl.pallas_call(
        _scale_body,
        grid=(NB,),
        in_specs=[
            pl.BlockSpec((BN, D), lambda i: (i, 0)),
            pl.BlockSpec((H, D), lambda i: (0, 0)),
            pl.BlockSpec((NC, N_PAD), lambda i: (0, 0)),
        ],
        out_specs=pl.BlockSpec((BN, H), lambda i: (i, 0)),
        out_shape=jax.ShapeDtypeStruct((N, H), jnp.float32),
        name="gcn_scale_tc",
    )(x, W1, deg2)

    agg_parts = pl.kernel(
        _agg_body,
        out_type=jax.ShapeDtypeStruct((NC, N_PAD, H), jnp.float32),
        mesh=mesh,
        scratch_types=[
            pltpu.VMEM_SHARED((ACC_ROWS, H), jnp.float32),
            pltpu.VMEM((NBUF, CH, H), jnp.float32),
            pltpu.VMEM((GBLK, CH), jnp.int32),
            pltpu.VMEM((GBLK, CH), jnp.int32),
            pltpu.SemaphoreType.DMA,
            pltpu.SemaphoreType.DMA,
            pltpu.SemaphoreType.DMA,
            pltpu.SemaphoreType.DMA,
        ],
        name="gcn_agg_sc",
    )(g, ei_p)

    y = pl.pallas_call(
        _head_body,
        grid=(NB,),
        in_specs=[
            pl.BlockSpec((NC, BN, H), lambda i: (0, i, 0)),
            pl.BlockSpec((BN, H), lambda i: (i, 0)),
            pl.BlockSpec((NC, N_PAD), lambda i: (0, 0)),
            pl.BlockSpec((1, H), lambda i: (0, 0)),
            pl.BlockSpec((1, H), lambda i: (0, 0)),
            pl.BlockSpec((1, 1), lambda i: (0, 0)),
        ],
        out_specs=pl.BlockSpec((1, BN), lambda i: (0, i)),
        out_shape=jax.ShapeDtypeStruct((1, N), jnp.float32),
        name="gcn_head_tc",
    )(agg_parts, g, deg2, b1.reshape(1, H), Wl, bl.reshape(1, 1))

    return y[0]


# speedup vs baseline: 1.2085x; 1.1092x over previous
import functools

import jax
import jax.numpy as jnp
from jax import lax
from jax.experimental import pallas as pl
from jax.experimental.pallas import tpu as pltpu
from jax.experimental.pallas import tpu_sc as plsc

N = 10000
D = 128
H = 128
E = 320000

NC = 2
NS = 16
NW = NC * NS

CH = 128
CPT = 80
NBUF = 2
GBLK = 16
EPT = CPT * CH
E_PAD = NW * EPT
ACC_ROWS = 10240
ROWS_PER_TILE = ACC_ROWS // NS
N_PAD = ACC_ROWS


def _zero_f32_vmem2(ref, n_rows, n_cols):
    z = jnp.zeros((16,), jnp.float32)

    def body(r, _):
        for c in range(n_cols // 16):
            ref[r, pl.ds(c * 16, 16)] = z
        return 0

    lax.fori_loop(0, n_rows, body, 0)


def _zero_f32_vmem1(ref, n):
    z = jnp.zeros((16,), jnp.float32)

    def body(i, _):
        ref[pl.ds(i * 16, 16)] = z
        return 0

    lax.fori_loop(0, n // 16, body, 0)


def _deg_body(ei_hbm, out_hbm, acc, ones_v, didx2, zbuf_v, dsem):
    c = lax.axis_index("c")
    s = lax.axis_index("s")
    wid = c * NS + s

    _zero_f32_vmem1(zbuf_v, ROWS_PER_TILE)
    pltpu.sync_copy(zbuf_v, acc.at[pl.ds(s * ROWS_PER_TILE, ROWS_PER_TILE)])
    o = jnp.full((16,), 1.0, jnp.float32)
    for c16 in range(CH // 16):
        ones_v[pl.ds(c16 * 16, 16)] = o
    pltpu.sync_copy(ei_hbm.at[1, wid], didx2)
    plsc.subcore_barrier()

    def chunk(k, carry):
        pltpu.async_copy(ones_v, acc.at[didx2.at[k]], dsem, add=True)
        return carry

    lax.fori_loop(0, CPT, chunk, 0)

    def drainc(k, carry):
        pltpu.make_async_copy(ones_v, acc.at[didx2.at[k]], dsem).wait()
        return carry

    lax.fori_loop(0, CPT, drainc, 0)
    plsc.subcore_barrier()

    pltpu.sync_copy(acc.at[pl.ds(s * ROWS_PER_TILE, ROWS_PER_TILE)],
                    out_hbm.at[c, pl.ds(s * ROWS_PER_TILE, ROWS_PER_TILE)])


def _agg_body(g_hbm, ei_hbm, out_hbm, acc, rows2, sidxb, didxb, zbuf,
              sem0, sem1, zsem, isem0, isem1):
    c = lax.axis_index("c")
    s = lax.axis_index("s")
    wid = c * NS + s
    sems = [sem0, sem1]
    NBLK = CPT // GBLK

    z16 = jnp.zeros((16,), jnp.float32)
    for r in range(zbuf.shape[0]):
        for cc in range(H // 16):
            zbuf[r, pl.ds(cc * 16, 16)] = z16
    ZR = zbuf.shape[0]
    for z in range(ROWS_PER_TILE // ZR):
        pltpu.async_copy(zbuf, acc.at[pl.ds(s * ROWS_PER_TILE + z * ZR, ZR)],
                         zsem)
    pltpu.sync_copy(ei_hbm.at[0, wid, pl.ds(0, GBLK)], sidxb.at[0])
    pltpu.sync_copy(ei_hbm.at[1, wid, pl.ds(0, GBLK)], didxb.at[0])
    for z in range(ROWS_PER_TILE // ZR):
        pltpu.make_async_copy(zbuf, acc.at[pl.ds(0, ZR)], zsem).wait()
    plsc.subcore_barrier()

    def issue(p, kk, b):
        pltpu.async_copy(g_hbm.at[sidxb.at[p, kk]], rows2.at[b], sems[b])

    def drain(p, kk, b):
        pltpu.make_async_copy(g_hbm.at[sidxb.at[p, kk]], rows2.at[b],
                              sems[b]).wait()

    def step(p, k, b):
        drain(p, k, b)
        pltpu.sync_copy(rows2.at[b], acc.at[didxb.at[p, k]], add=True)

    issue(0, 0, 0)
    for j in range(NBLK):
        p = j % 2
        pn = (j + 1) % 2
        if j + 1 < NBLK:
            pltpu.async_copy(ei_hbm.at[0, wid, pl.ds((j + 1) * GBLK, GBLK)],
                             sidxb.at[pn], isem0)
            pltpu.async_copy(ei_hbm.at[1, wid, pl.ds((j + 1) * GBLK, GBLK)],
                             didxb.at[pn], isem1)

        def pair(i, _):
            for b in range(NBUF):
                k = NBUF * i + b
                issue(p, k + 1, (b + 1) % NBUF)
                step(p, k, b)
            return 0

        lax.fori_loop(0, GBLK // NBUF - 1, pair, 0)
        kl = GBLK - 2
        issue(p, kl + 1, 1)
        step(p, kl, 0)
        if j + 1 < NBLK:
            pltpu.make_async_copy(ei_hbm.at[0, wid, pl.ds((j + 1) * GBLK, GBLK)],
                                  sidxb.at[pn], isem0).wait()
            pltpu.make_async_copy(ei_hbm.at[1, wid, pl.ds((j + 1) * GBLK, GBLK)],
                                  didxb.at[pn], isem1).wait()
            issue(pn, 0, 0)
        step(p, kl + 1, 1)
    plsc.subcore_barrier()

    pltpu.sync_copy(acc.at[pl.ds(s * ROWS_PER_TILE, ROWS_PER_TILE)],
                    out_hbm.at[c, pl.ds(s * ROWS_PER_TILE, ROWS_PER_TILE)])


def _scale_body(x_ref, w1_ref, deg_ref, g_ref):
    i = pl.program_id(0)
    bn = g_ref.shape[0]
    deg = (deg_ref[0, pl.ds(i * bn, bn)] + deg_ref[1, pl.ds(i * bn, bn)] + 1.0)
    dinv = 1.0 / jnp.sqrt(deg)
    h = lax.dot_general(x_ref[...], w1_ref[...], (((1,), (1,)), ((), ())),
                        preferred_element_type=jnp.float32)
    g_ref[...] = h * dinv[:, None]


def _head_body(agg_ref, g_ref, deg_ref, b1_ref, wl_ref, bl_ref, y_ref):
    i = pl.program_id(0)
    bn = g_ref.shape[0]
    deg = (deg_ref[0, pl.ds(i * bn, bn)] + deg_ref[1, pl.ds(i * bn, bn)] + 1.0)
    dinv = 1.0 / jnp.sqrt(deg)
    tot = agg_ref[0] + agg_ref[1] + g_ref[...]
    conv = tot * dinv[:, None] + b1_ref[0, :][None, :]
    conv = jnp.maximum(conv, 0.0)
    y = lax.dot_general(wl_ref[...], conv, (((1,), (1,)), ((), ())),
                        preferred_element_type=jnp.float32)
    y_ref[...] = y + bl_ref[0, 0]


def kernel(x, edge_index, W1, b1, Wl, bl):
    pad = E_PAD - E
    ar = jnp.arange(pad, dtype=jnp.int32)
    pad_pair = jnp.stack([ar % N, N + ar % (ACC_ROWS - N)])
    ei_p = jnp.concatenate([edge_index, pad_pair], axis=1)
    ei_p = ei_p.reshape(2, NW, CPT, CH)

    mesh = plsc.VectorSubcoreMesh(core_axis_name="c", subcore_axis_name="s")

    deg_parts = pl.kernel(
        _deg_body,
        out_type=jax.ShapeDtypeStruct((NC, ACC_ROWS), jnp.float32),
        mesh=mesh,
        scratch_types=[
            pltpu.VMEM_SHARED((ACC_ROWS,), jnp.float32),
            pltpu.VMEM((CH,), jnp.float32),
            pltpu.VMEM((CPT, CH), jnp.int32),
            pltpu.VMEM((ROWS_PER_TILE,), jnp.float32),
            pltpu.SemaphoreType.DMA,
        ],
        name="gcn_deg_sc",
    )(ei_p)

    deg2 = deg_parts

    BN = 2048
    NB = (N + BN - 1) // BN
    g = pl.pallas_call(
        _scale_body,
        grid=(NB,),
        in_specs=[
            pl.BlockSpec((BN, D), lambda i: (i, 0)),
            pl.BlockSpec((H, D), lambda i: (0, 0)),
            pl.BlockSpec((NC, N_PAD), lambda i: (0, 0)),
        ],
        out_specs=pl.BlockSpec((BN, H), lambda i: (i, 0)),
        out_shape=jax.ShapeDtypeStruct((N, H), jnp.float32),
        name="gcn_scale_tc",
    )(x, W1, deg2)

    agg_parts = pl.kernel(
        _agg_body,
        out_type=jax.ShapeDtypeStruct((NC, N_PAD, H), jnp.float32),
        mesh=mesh,
        scratch_types=[
            pltpu.VMEM_SHARED((ACC_ROWS, H), jnp.float32),
            pltpu.VMEM((NBUF, CH, H), jnp.float32),
            pltpu.VMEM((2, GBLK, CH), jnp.int32),
            pltpu.VMEM((2, GBLK, CH), jnp.int32),
            pltpu.VMEM((16, H), jnp.float32),
            pltpu.SemaphoreType.DMA,
            pltpu.SemaphoreType.DMA,
            pltpu.SemaphoreType.DMA,
            pltpu.SemaphoreType.DMA,
            pltpu.SemaphoreType.DMA,
        ],
        name="gcn_agg_sc",
    )(g, ei_p)

    y = pl.pallas_call(
        _head_body,
        grid=(NB,),
        in_specs=[
            pl.BlockSpec((NC, BN, H), lambda i: (0, i, 0)),
            pl.BlockSpec((BN, H), lambda i: (i, 0)),
            pl.BlockSpec((NC, N_PAD), lambda i: (0, 0)),
            pl.BlockSpec((1, H), lambda i: (0, 0)),
            pl.BlockSpec((1, H), lambda i: (0, 0)),
            pl.BlockSpec((1, 1), lambda i: (0, 0)),
        ],
        out_specs=pl.BlockSpec((1, BN), lambda i: (0, i)),
        out_shape=jax.ShapeDtypeStruct((1, N), jnp.float32),
        name="gcn_head_tc",
    )(agg_parts, g, deg2, b1.reshape(1, H), Wl, bl.reshape(1, 1))

    return y[0]


# speedup vs baseline: 1.2128x; 1.0035x over previous
import jax
import jax.numpy as jnp
from jax import lax
from jax.experimental import pallas as pl
from jax.experimental.pallas import tpu as pltpu
from jax.experimental.pallas import tpu_sc as plsc

N = 10000
D = 128
H = 128
E = 320000

NC = 2
NS = 16
NW = NC * NS

CH = 128
CPT = 80
NBUF = 2
GBLK = 16
EPT = CPT * CH
E_PAD = NW * EPT
ACC_ROWS = 10240
ROWS_PER_TILE = ACC_ROWS // NS
N_PAD = ACC_ROWS


def _zero_f32_vmem1(ref, n):
    z = jnp.zeros((16,), jnp.float32)

    def body(i, _):
        ref[pl.ds(i * 16, 16)] = z
        return 0

    lax.fori_loop(0, n // 16, body, 0)


def _deg_body(ei_hbm, out_hbm, acc, ones_v, didx2, zbuf_v, dsem):
    c = lax.axis_index("c")
    s = lax.axis_index("s")
    wid = c * NS + s

    _zero_f32_vmem1(zbuf_v, ROWS_PER_TILE)
    pltpu.sync_copy(zbuf_v, acc.at[pl.ds(s * ROWS_PER_TILE, ROWS_PER_TILE)])
    o = jnp.full((16,), 1.0, jnp.float32)
    for c16 in range(CH // 16):
        ones_v[pl.ds(c16 * 16, 16)] = o
    pltpu.sync_copy(ei_hbm.at[1, wid], didx2)
    plsc.subcore_barrier()

    def chunk(k, carry):
        pltpu.async_copy(ones_v, acc.at[didx2.at[k]], dsem, add=True)
        return carry

    lax.fori_loop(0, CPT, chunk, 0)

    def drainc(k, carry):
        pltpu.make_async_copy(ones_v, acc.at[didx2.at[k]], dsem).wait()
        return carry

    lax.fori_loop(0, CPT, drainc, 0)
    plsc.subcore_barrier()

    pltpu.sync_copy(acc.at[pl.ds(s * ROWS_PER_TILE, ROWS_PER_TILE)],
                    out_hbm.at[c, pl.ds(s * ROWS_PER_TILE, ROWS_PER_TILE)])


def _agg_body(g_hbm, ei_hbm, out_hbm, acc, rows2, sidxb, didxb, zbuf,
              sem0, sem1, zsem, isem0, isem1):
    c = lax.axis_index("c")
    s = lax.axis_index("s")
    wid = c * NS + s
    sems = [sem0, sem1]
    NBLK = CPT // GBLK

    z16 = jnp.zeros((16,), jnp.float32)
    for r in range(zbuf.shape[0]):
        for cc in range(H // 16):
            zbuf[r, pl.ds(cc * 16, 16)] = z16
    ZR = zbuf.shape[0]
    for z in range(ROWS_PER_TILE // ZR):
        pltpu.async_copy(zbuf, acc.at[pl.ds(s * ROWS_PER_TILE + z * ZR, ZR)],
                         zsem)
    pltpu.sync_copy(ei_hbm.at[0, wid, pl.ds(0, GBLK)], sidxb.at[0])
    pltpu.sync_copy(ei_hbm.at[1, wid, pl.ds(0, GBLK)], didxb.at[0])
    for z in range(ROWS_PER_TILE // ZR):
        pltpu.make_async_copy(zbuf, acc.at[pl.ds(0, ZR)], zsem).wait()
    plsc.subcore_barrier()

    def issue(p, kk, b):
        pltpu.async_copy(g_hbm.at[sidxb.at[p, kk]], rows2.at[b], sems[b])

    def drain(p, kk, b):
        pltpu.make_async_copy(g_hbm.at[sidxb.at[p, kk]], rows2.at[b],
                              sems[b]).wait()

    def step(p, k, b):
        drain(p, k, b)
        pltpu.sync_copy(rows2.at[b], acc.at[didxb.at[p, k]], add=True)

    issue(0, 0, 0)
    for j in range(NBLK):
        p = j % 2
        pn = (j + 1) % 2
        if j + 1 < NBLK:
            pltpu.async_copy(ei_hbm.at[0, wid, pl.ds((j + 1) * GBLK, GBLK)],
                             sidxb.at[pn], isem0)
            pltpu.async_copy(ei_hbm.at[1, wid, pl.ds((j + 1) * GBLK, GBLK)],
                             didxb.at[pn], isem1)

        def pair(i, _):
            for b in range(NBUF):
                k = NBUF * i + b
                issue(p, k + 1, (b + 1) % NBUF)
                step(p, k, b)
            return 0

        lax.fori_loop(0, GBLK // NBUF - 1, pair, 0)
        kl = GBLK - 2
        issue(p, kl + 1, 1)
        step(p, kl, 0)
        if j + 1 < NBLK:
            pltpu.make_async_copy(ei_hbm.at[0, wid, pl.ds((j + 1) * GBLK, GBLK)],
                                  sidxb.at[pn], isem0).wait()
            pltpu.make_async_copy(ei_hbm.at[1, wid, pl.ds((j + 1) * GBLK, GBLK)],
                                  didxb.at[pn], isem1).wait()
            issue(pn, 0, 0)
        step(p, kl + 1, 1)
    plsc.subcore_barrier()

    pltpu.sync_copy(acc.at[pl.ds(s * ROWS_PER_TILE, ROWS_PER_TILE)],
                    out_hbm.at[c, pl.ds(s * ROWS_PER_TILE, ROWS_PER_TILE)])


def _scale_body(x_ref, w1_ref, deg_ref, g_ref):
    i = pl.program_id(0)
    bn = g_ref.shape[0]
    deg = (deg_ref[0, pl.ds(i * bn, bn)] + deg_ref[1, pl.ds(i * bn, bn)] + 1.0)
    dinv = 1.0 / jnp.sqrt(deg)
    h = lax.dot_general(x_ref[...], w1_ref[...], (((1,), (1,)), ((), ())),
                        preferred_element_type=jnp.float32)
    g_ref[...] = h * dinv[:, None]


def _head_body(agg_ref, g_ref, deg_ref, b1_ref, wl_ref, bl_ref, y_ref):
    i = pl.program_id(0)
    bn = g_ref.shape[0]
    deg = (deg_ref[0, pl.ds(i * bn, bn)] + deg_ref[1, pl.ds(i * bn, bn)] + 1.0)
    dinv = 1.0 / jnp.sqrt(deg)
    tot = agg_ref[0] + agg_ref[1] + g_ref[...]
    conv = tot * dinv[:, None] + b1_ref[0, :][None, :]
    conv = jnp.maximum(conv, 0.0)
    y = lax.dot_general(wl_ref[...], conv, (((1,), (1,)), ((), ())),
                        preferred_element_type=jnp.float32)
    y_ref[...] = y + bl_ref[0, 0]


def kernel(x, edge_index, W1, b1, Wl, bl):
    pad = E_PAD - E
    ar = jnp.arange(pad, dtype=jnp.int32)
    pad_pair = jnp.stack([ar % N, N + ar % (ACC_ROWS - N)])
    ei_p = jnp.concatenate([edge_index, pad_pair], axis=1)
    ei_p = ei_p.reshape(2, NW, CPT, CH)

    mesh = plsc.VectorSubcoreMesh(core_axis_name="c", subcore_axis_name="s")

    deg_parts = pl.kernel(
        _deg_body,
        out_type=jax.ShapeDtypeStruct((NC, ACC_ROWS), jnp.float32),
        mesh=mesh,
        scratch_types=[
            pltpu.VMEM_SHARED((ACC_ROWS,), jnp.float32),
            pltpu.VMEM((CH,), jnp.float32),
            pltpu.VMEM((CPT, CH), jnp.int32),
            pltpu.VMEM((ROWS_PER_TILE,), jnp.float32),
            pltpu.SemaphoreType.DMA,
        ],
        name="gcn_deg_sc",
    )(ei_p)

    deg2 = deg_parts

    BN = 2048
    NB = (N + BN - 1) // BN
    g = pl.pallas_call(
        _scale_body,
        grid=(NB,),
        in_specs=[
            pl.BlockSpec((BN, D), lambda i: (i, 0)),
            pl.BlockSpec((H, D), lambda i: (0, 0)),
            pl.BlockSpec((NC, N_PAD), lambda i: (0, 0)),
        ],
        out_specs=pl.BlockSpec((BN, H), lambda i: (i, 0)),
        out_shape=jax.ShapeDtypeStruct((N, H), jnp.float32),
        name="gcn_scale_tc",
    )(x, W1, deg2)

    agg_parts = pl.kernel(
        _agg_body,
        out_type=jax.ShapeDtypeStruct((NC, N_PAD, H), jnp.float32),
        mesh=mesh,
        scratch_types=[
            pltpu.VMEM_SHARED((ACC_ROWS, H), jnp.float32),
            pltpu.VMEM((NBUF, CH, H), jnp.float32),
            pltpu.VMEM((2, GBLK, CH), jnp.int32),
            pltpu.VMEM((2, GBLK, CH), jnp.int32),
            pltpu.VMEM((16, H), jnp.float32),
            pltpu.SemaphoreType.DMA,
            pltpu.SemaphoreType.DMA,
            pltpu.SemaphoreType.DMA,
            pltpu.SemaphoreType.DMA,
            pltpu.SemaphoreType.DMA,
        ],
        name="gcn_agg_sc",
    )(g, ei_p)

    y = pl.pallas_call(
        _head_body,
        grid=(NB,),
        in_specs=[
            pl.BlockSpec((NC, BN, H), lambda i: (0, i, 0)),
            pl.BlockSpec((BN, H), lambda i: (i, 0)),
            pl.BlockSpec((NC, N_PAD), lambda i: (0, 0)),
            pl.BlockSpec((1, H), lambda i: (0, 0)),
            pl.BlockSpec((1, H), lambda i: (0, 0)),
            pl.BlockSpec((1, 1), lambda i: (0, 0)),
        ],
        out_specs=pl.BlockSpec((1, BN), lambda i: (0, i)),
        out_shape=jax.ShapeDtypeStruct((1, N), jnp.float32),
        name="gcn_head_tc",
    )(agg_parts, g, deg2, b1.reshape(1, H), Wl, bl.reshape(1, 1))

    return y[0]
